# baseline (speedup 1.0000x reference)
import functools

import jax
import jax.numpy as jnp
from jax import lax
from jax.experimental import pallas as pl
from jax.experimental.pallas import tpu as pltpu
from jax.experimental.pallas import tpu_sc as plsc

_B = 16
_MAX_LEN = 2048
_TOTAL = 16384
_LANES = 16
_W = _MAX_LEN + _LANES
_CHUNKS = _MAX_LEN // _LANES


_UNROLL = 8
_HALF = _MAX_LEN // 2


def _emit_row(val_hbm, out_hbm, win_v, row_v, astart, shift_v, len_v16, sub,
              sem0, sem1):
    pltpu.sync_copy(val_hbm.at[pl.ds(astart, _W)], win_v)
    iota = lax.iota(jnp.int32, _LANES)
    zero = jnp.zeros((_LANES,), dtype=val_hbm.dtype)

    def make_group(half):
        def group(g, _):
            base = half * _HALF + g * (_LANES * _UNROLL)
            for u in range(_UNROLL):
                pos = iota + (base + u * _LANES)
                mask = pos < len_v16
                vals = plsc.load_gather(win_v, [pos + shift_v], mask=mask)
                off = pl.multiple_of(base + u * _LANES, _LANES)
                row_v[pl.ds(off, _LANES)] = jnp.where(mask, vals, zero)
            return 0
        return group

    groups_per_half = _HALF // (_LANES * _UNROLL)
    lax.fori_loop(0, groups_per_half, make_group(0), 0)
    cp0 = pltpu.make_async_copy(row_v.at[pl.ds(0, _HALF)],
                                out_hbm.at[sub, pl.ds(0, _HALF)], sem0)
    cp0.start()
    lax.fori_loop(0, groups_per_half, make_group(1), 0)
    cp1 = pltpu.make_async_copy(row_v.at[pl.ds(_HALF, _HALF)],
                                out_hbm.at[sub, pl.ds(_HALF, _HALF)], sem1)
    cp1.start()
    cp0.wait()
    cp1.wait()


def _body(item_hbm, price_hbm, off_hbm, out_item, out_price, out_li, out_lp,
          off_v, win_i, win_f, row_i, row_f, len_v, sem0, sem1):
    core = lax.axis_index("c")
    sub = lax.axis_index("s")
    pltpu.sync_copy(off_hbm, off_v)
    sub_splat = jnp.full((_LANES,), sub, dtype=jnp.int32)
    start_v = plsc.load_gather(off_v, [sub_splat])
    end_v = plsc.load_gather(off_v, [sub_splat + 1])
    len_v16 = end_v - start_v
    astart_v = jnp.minimum(jnp.bitwise_and(start_v, -_LANES), _TOTAL - _W)
    shift_v = start_v - astart_v
    astart = pl.multiple_of(astart_v[0], _LANES)

    @pl.when(core == 0)
    def _():
        _emit_row(item_hbm, out_item, win_i, row_i, astart, shift_v, len_v16,
                  sub, sem0, sem1)

    @pl.when(core == 1)
    def _():
        _emit_row(price_hbm, out_price, win_f, row_f, astart, shift_v, len_v16,
                  sub, sem0, sem1)

    @pl.when(sub == 0)
    def _():
        iota = lax.iota(jnp.int32, _LANES)
        lo = plsc.load_gather(off_v, [iota])
        hi = plsc.load_gather(off_v, [iota + 1])
        len_v[...] = hi - lo

        @pl.when(core == 0)
        def _():
            pltpu.sync_copy(len_v, out_li)

        @pl.when(core == 1)
        def _():
            pltpu.sync_copy(len_v, out_lp)


@jax.jit
def _padder(item_i32, price_f32, off_pad):
    mesh = plsc.VectorSubcoreMesh(core_axis_name="c", subcore_axis_name="s")
    return pl.kernel(
        _body,
        out_type=[
            jax.ShapeDtypeStruct((_B, _MAX_LEN), jnp.int32),
            jax.ShapeDtypeStruct((_B, _MAX_LEN), jnp.float32),
            jax.ShapeDtypeStruct((_B,), jnp.int32),
            jax.ShapeDtypeStruct((_B,), jnp.int32),
        ],
        mesh=mesh,
        scratch_types=[
            pltpu.VMEM((_B + 1,), jnp.int32),
            pltpu.VMEM((_W,), jnp.int32),
            pltpu.VMEM((_W,), jnp.float32),
            pltpu.VMEM((_MAX_LEN,), jnp.int32),
            pltpu.VMEM((_MAX_LEN,), jnp.float32),
            pltpu.VMEM((_LANES,), jnp.int32),
            pltpu.SemaphoreType.DMA,
            pltpu.SemaphoreType.DMA,
        ],
        compiler_params=pltpu.CompilerParams(needs_layout_passes=False),
        name="tabular_padding_sc",
    )(item_i32, price_f32, off_pad)


def kernel(item_id_values, price_values, offsets):
    item_i32 = item_id_values.astype(jnp.int32)
    price_f32 = price_values.astype(jnp.float32)
    off_i32 = offsets.astype(jnp.int32)
    padded_item, padded_price, len_i, len_p = _padder(item_i32, price_f32, off_i32)
    out_dtype = item_id_values.dtype
    len_dtype = offsets.dtype
    return (padded_item.astype(out_dtype), padded_price,
            len_i.astype(len_dtype), len_p.astype(len_dtype))


# speedup vs baseline: 1.0544x; 1.0544x over previous
import functools

import jax
import jax.numpy as jnp
from jax import lax
from jax.experimental import pallas as pl
from jax.experimental.pallas import tpu as pltpu
from jax.experimental.pallas import tpu_sc as plsc

_B = 16
_MAX_LEN = 2048
_TOTAL = 16384
_LANES = 16
_W = _MAX_LEN + _LANES
_CHUNKS = _MAX_LEN // _LANES


_UNROLL = 8


def _emit_row(val_hbm, out_hbm, win_v, row_v, astart, shift_v, len_v16, sub):
    pltpu.sync_copy(val_hbm.at[pl.ds(astart, _W)], win_v)
    iota = lax.iota(jnp.int32, _LANES)
    zero = jnp.zeros((_LANES,), dtype=val_hbm.dtype)

    def group(g, _):
        base = g * (_LANES * _UNROLL)
        for u in range(_UNROLL):
            pos = iota + (base + u * _LANES)
            mask = pos < len_v16
            vals = plsc.load_gather(win_v, [pos + shift_v], mask=mask)
            off = pl.multiple_of(base + u * _LANES, _LANES)
            row_v[pl.ds(off, _LANES)] = jnp.where(mask, vals, zero)
        return 0

    lax.fori_loop(0, _CHUNKS // _UNROLL, group, 0)
    pltpu.sync_copy(row_v, out_hbm.at[sub])


def _body(item_hbm, price_hbm, off_hbm, out_item, out_price, out_li, out_lp,
          off_v, win_i, win_f, row_i, row_f, len_v):
    core = lax.axis_index("c")
    sub = lax.axis_index("s")
    pltpu.sync_copy(off_hbm, off_v)
    sub_splat = jnp.full((_LANES,), sub, dtype=jnp.int32)
    start_v = plsc.load_gather(off_v, [sub_splat])
    end_v = plsc.load_gather(off_v, [sub_splat + 1])
    len_v16 = end_v - start_v
    astart_v = jnp.minimum(jnp.bitwise_and(start_v, -_LANES), _TOTAL - _W)
    shift_v = start_v - astart_v
    astart = pl.multiple_of(astart_v[0], _LANES)

    @pl.when(core == 0)
    def _():
        _emit_row(item_hbm, out_item, win_i, row_i, astart, shift_v, len_v16, sub)

    @pl.when(core == 1)
    def _():
        _emit_row(price_hbm, out_price, win_f, row_f, astart, shift_v, len_v16, sub)

    @pl.when(sub == 0)
    def _():
        iota = lax.iota(jnp.int32, _LANES)
        lo = plsc.load_gather(off_v, [iota])
        hi = plsc.load_gather(off_v, [iota + 1])
        len_v[...] = hi - lo

        @pl.when(core == 0)
        def _():
            pltpu.sync_copy(len_v, out_li)

        @pl.when(core == 1)
        def _():
            pltpu.sync_copy(len_v, out_lp)


@jax.jit
def _padder(item_i32, price_f32, off_pad):
    mesh = plsc.VectorSubcoreMesh(core_axis_name="c", subcore_axis_name="s")
    return pl.kernel(
        _body,
        out_type=[
            jax.ShapeDtypeStruct((_B, _MAX_LEN), jnp.int32),
            jax.ShapeDtypeStruct((_B, _MAX_LEN), jnp.float32),
            jax.ShapeDtypeStruct((_B,), jnp.int32),
            jax.ShapeDtypeStruct((_B,), jnp.int32),
        ],
        mesh=mesh,
        scratch_types=[
            pltpu.VMEM((_B + 1,), jnp.int32),
            pltpu.VMEM((_W,), jnp.int32),
            pltpu.VMEM((_W,), jnp.float32),
            pltpu.VMEM((_MAX_LEN,), jnp.int32),
            pltpu.VMEM((_MAX_LEN,), jnp.float32),
            pltpu.VMEM((_LANES,), jnp.int32),
        ],
        compiler_params=pltpu.CompilerParams(
            needs_layout_passes=False,
            disable_bounds_checks=True,
            disable_semaphore_checks=True,
        ),
        name="tabular_padding_sc",
    )(item_i32, price_f32, off_pad)


def kernel(item_id_values, price_values, offsets):
    item_i32 = item_id_values.astype(jnp.int32)
    price_f32 = price_values.astype(jnp.float32)
    off_i32 = offsets.astype(jnp.int32)
    padded_item, padded_price, len_i, len_p = _padder(item_i32, price_f32, off_i32)
    out_dtype = item_id_values.dtype
    len_dtype = offsets.dtype
    return (padded_item.astype(out_dtype), padded_price,
            len_i.astype(len_dtype), len_p.astype(len_dtype))
